# Initial kernel scaffold; baseline (speedup 1.0000x reference)
import jax
import jax.numpy as jnp
from jax.experimental import pallas as pl


def kernel(x, edge_index, edge_attr, Wq1, bq1, Wk1, bk1, Wv1, bv1, We1, Wskip1, bskip1, Wq2, bq2, Wk2, bk2, Wv2, bv2, We2, Wskip2, bskip2, W3, b3, W4, b4):
    raise NotImplementedError("write your pallas kernel here")


# speedup vs baseline: 64.2126x; 64.2126x over previous
import functools

import jax
import jax.numpy as jnp
from jax import lax
from jax.experimental import pallas as pl
from jax.experimental.pallas import tpu as pltpu
from jax.experimental.pallas import tpu_sc as plsc

N = 10000
E = 320000
D_IN = 128
D_EDGE = 16
HID = 16
HEADS = 5
N_CLASSES = 2

SHIFT = 8.0

NC = 2
NS = 16
LANES = 16
NW = NC * NS

B = 128
NBLK = E // B
BLK_PER_TILE = -(-NBLK // NW)

f32 = jnp.float32


def _tc1a_body(x_ref, wq_ref, bq_ref, wk_ref, bk_ref, wv_ref, bv_ref,
               wsk_ref, bsk_ref, qs_ref, kv_ref, skip_ref):
    x = x_ref[...]
    q = (jnp.dot(x, wq_ref[...], preferred_element_type=f32) + bq_ref[...]) * 0.25
    k = jnp.dot(x, wk_ref[...], preferred_element_type=f32) + bk_ref[...]
    v = jnp.dot(x, wv_ref[...], preferred_element_type=f32) + bv_ref[...]
    qs_ref[...] = q
    kv_ref[...] = jnp.concatenate([k, v], axis=1)
    skip_ref[...] = jnp.dot(x, wsk_ref[...], preferred_element_type=f32) + bsk_ref[...]


def _tc1a(x, Wq, bq, Wk, bk, Wv, bv, Wsk, bsk):
    R = 1000
    grid = (N // R,)
    full = lambda a: pl.BlockSpec(a.shape, lambda i: (0,) * a.ndim)
    return pl.pallas_call(
        _tc1a_body,
        grid=grid,
        in_specs=[pl.BlockSpec((R, D_IN), lambda i: (i, 0)),
                  full(Wq), full(bq), full(Wk), full(bk), full(Wv), full(bv),
                  full(Wsk), full(bsk)],
        out_specs=[pl.BlockSpec((R, 80), lambda i: (i, 0)),
                   pl.BlockSpec((R, 160), lambda i: (i, 0)),
                   pl.BlockSpec((R, HID), lambda i: (i, 0))],
        out_shape=[jax.ShapeDtypeStruct((N, 80), f32),
                   jax.ShapeDtypeStruct((N, 160), f32),
                   jax.ShapeDtypeStruct((N, HID), f32)],
    )(x, Wq, bq, Wk, bk, Wv, bv, Wsk, bsk)


def _tc1b_body(ea_ref, we1_ref, we2_ref, e1_ref, e2_ref):
    ea = ea_ref[...]
    e1_ref[...] = jnp.dot(ea, we1_ref[...], preferred_element_type=f32)
    e2 = jnp.dot(ea, we2_ref[...], preferred_element_type=f32)
    r = e2.shape[0]
    e2_ref[...] = jnp.concatenate([e2, jnp.zeros((r, 8), f32)], axis=1)


def _tc1b(edge_attr, We1, We2):
    R = 4000
    grid = (E // R,)
    full = lambda a: pl.BlockSpec(a.shape, lambda i: (0,) * a.ndim)
    return pl.pallas_call(
        _tc1b_body,
        grid=grid,
        in_specs=[pl.BlockSpec((R, D_EDGE), lambda i: (i, 0)),
                  full(We1), full(We2)],
        out_specs=[pl.BlockSpec((R, 80), lambda i: (i, 0)),
                   pl.BlockSpec((R, 48), lambda i: (i, 0))],
        out_shape=[jax.ShapeDtypeStruct((E, 80), f32),
                   jax.ShapeDtypeStruct((E, 48), f32)],
    )(edge_attr, We1, We2)


def _sc_edge_pass(src, dst, qtab, kvtab, etab, wq, wacc, packed):
    nj = wq // LANES
    mesh = plsc.VectorSubcoreMesh(core_axis_name="c", subcore_axis_name="s",
                                  num_cores=NC, num_subcores=NS)

    @functools.partial(
        pl.kernel,
        out_type=jax.ShapeDtypeStruct((NC, N, wacc), f32),
        mesh=mesh,
        scratch_types=[
            pltpu.VMEM((B,), jnp.int32),
            pltpu.VMEM((B,), jnp.int32),
            pltpu.VMEM((B, wq), f32),
            pltpu.VMEM((B, 2 * wq), f32),
            pltpu.VMEM((B, wq), f32),
            pltpu.VMEM((B, wacc), f32),
            pltpu.VMEM_SHARED((N, wacc), f32),
            pltpu.VMEM((B, wacc), f32),
            pltpu.SemaphoreType.DMA,
            pltpu.SemaphoreType.DMA,
            pltpu.SemaphoreType.DMA,
        ],
        compiler_params=pltpu.CompilerParams(needs_layout_passes=False,
                                             use_tc_tiling_on_sc=False),
    )
    def body(src_hbm, dst_hbm, q_hbm, kv_hbm, e_hbm, out_hbm,
             srcv, dstv, qr, kvr, er, outr, acc, zbuf, sem0, sem1, sem2):
        cid = lax.axis_index("c")
        sid = lax.axis_index("s")
        wid = sid * NC + cid
        iota = lax.iota(jnp.int32, LANES)
        zeros = jnp.zeros((LANES,), f32)

        @pl.loop(0, B)
        def _zrow(r):
            for c in range(wacc // LANES):
                zbuf[r, pl.ds(LANES * c, LANES)] = zeros

        nfull = N // B
        ntail = N - nfull * B
        for t in range(-(-(nfull + 1) // NS)):
            ck = sid + NS * t

            @pl.when(ck < nfull)
            def _():
                pltpu.sync_copy(zbuf, acc.at[pl.ds(ck * B, B)])

            @pl.when(ck == nfull)
            def _():
                pltpu.sync_copy(zbuf.at[pl.ds(0, ntail)],
                                acc.at[pl.ds(nfull * B, ntail)])

        plsc.subcore_barrier()

        @pl.loop(0, BLK_PER_TILE)
        def _blk(j):
            blk = wid + NW * j

            @pl.when(blk < NBLK)
            def _():
                base = blk * B
                pltpu.sync_copy(src_hbm.at[pl.ds(base, B)], srcv)
                pltpu.sync_copy(dst_hbm.at[pl.ds(base, B)], dstv)
                cq = pltpu.async_copy(q_hbm.at[dstv], qr, sem0)
                ckv = pltpu.async_copy(kv_hbm.at[srcv], kvr, sem1)
                ce = pltpu.async_copy(e_hbm.at[pl.ds(base, B)], er, sem2)
                cq.wait()
                ckv.wait()
                ce.wait()

                @plsc.parallel_loop(0, B, unroll=2)
                def _edge(i):
                    den = zeros
                    for jh in range(nj):
                        sl = pl.ds(LANES * jh, LANES)
                        ev = er[i, sl]
                        kvec = kvr[i, sl] + ev
                        vvec = kvr[i, pl.ds(wq + LANES * jh, LANES)] + ev
                        p = qr[i, sl] * kvec
                        pre = plsc.cumsum(p)
                        if not packed:
                            a = jnp.full((LANES,), pre[LANES - 1], f32)
                            ex = jnp.exp(a - SHIFT)
                            outr[i, sl] = ex * vvec
                            den = jnp.where(iota == jh, ex, den)
                        else:
                            alo = jnp.full((LANES,), pre[7], f32)
                            ahi = jnp.full((LANES,), pre[LANES - 1], f32) - alo
                            av = jnp.where(iota < 8, alo, ahi)
                            ex = jnp.exp(av - SHIFT)
                            outr[i, sl] = ex * vvec
                            exlo = jnp.full((LANES,), ex[0], f32)
                            den = jnp.where(iota == 2 * jh, exlo, den)
                            if 2 * jh + 1 < HEADS:
                                exhi = jnp.full((LANES,), ex[8], f32)
                                den = jnp.where(iota == 2 * jh + 1, exhi, den)
                    outr[i, pl.ds(wq, LANES)] = den

                pltpu.sync_copy(outr, acc.at[dstv], add=True)

        plsc.subcore_barrier()

        for t in range(-(-(nfull + 1) // NS)):
            ck = sid + NS * t

            @pl.when(ck < nfull)
            def _():
                pltpu.sync_copy(acc.at[pl.ds(ck * B, B)],
                                out_hbm.at[cid, pl.ds(ck * B, B)])

            @pl.when(ck == nfull)
            def _():
                pltpu.sync_copy(acc.at[pl.ds(nfull * B, ntail)],
                                out_hbm.at[cid, pl.ds(nfull * B, ntail)])

    return body(src, dst, qtab, kvtab, etab)


def _tc2_body(acc_ref, skip_ref, wq_ref, bq_ref, wk_ref, bk_ref, wv_ref,
              bv_ref, wsk_ref, bsk_ref, qs_ref, kv_ref, skip2_ref):
    a = acc_ref[0] + acc_ref[1]
    r = a.shape[0]
    num = a[:, :80].reshape(r, HEADS, 16)
    den = a[:, 80:80 + HEADS]
    agg = num / (den[:, :, None] + 1e-30)
    h1 = jnp.maximum(jnp.mean(agg, axis=1) + skip_ref[...], 0.0)
    zpad = jnp.zeros((r, 8), f32)
    q = (jnp.dot(h1, wq_ref[...], preferred_element_type=f32) + bq_ref[...])
    qs_ref[...] = jnp.concatenate([q * (1.0 / jnp.sqrt(8.0)), zpad], axis=1)
    k = jnp.dot(h1, wk_ref[...], preferred_element_type=f32) + bk_ref[...]
    v = jnp.dot(h1, wv_ref[...], preferred_element_type=f32) + bv_ref[...]
    kv_ref[...] = jnp.concatenate([k, zpad, v, zpad], axis=1)
    skip2_ref[...] = jnp.dot(h1, wsk_ref[...], preferred_element_type=f32) + bsk_ref[...]


def _tc2(acc1, skip1, Wq, bq, Wk, bk, Wv, bv, Wsk, bsk):
    R = 1000
    grid = (N // R,)
    full = lambda a: pl.BlockSpec(a.shape, lambda i: (0,) * a.ndim)
    return pl.pallas_call(
        _tc2_body,
        grid=grid,
        in_specs=[pl.BlockSpec((NC, R, 96), lambda i: (0, i, 0)),
                  pl.BlockSpec((R, HID), lambda i: (i, 0)),
                  full(Wq), full(bq), full(Wk), full(bk), full(Wv), full(bv),
                  full(Wsk), full(bsk)],
        out_specs=[pl.BlockSpec((R, 48), lambda i: (i, 0)),
                   pl.BlockSpec((R, 96), lambda i: (i, 0)),
                   pl.BlockSpec((R, 8), lambda i: (i, 0))],
        out_shape=[jax.ShapeDtypeStruct((N, 48), f32),
                   jax.ShapeDtypeStruct((N, 96), f32),
                   jax.ShapeDtypeStruct((N, 8), f32)],
    )(acc1, skip1, Wq, bq, Wk, bk, Wv, bv, Wsk, bsk)


def _tc3_body(acc_ref, skip_ref, w3_ref, b3_ref, w4_ref, b4_ref, out_ref):
    a = acc_ref[0] + acc_ref[1]
    r = a.shape[0]
    num = a[:, :48].reshape(r, 6, 8)[:, :HEADS, :]
    den = a[:, 48:48 + HEADS]
    agg = num / (den[:, :, None] + 1e-30)
    h2 = jnp.maximum(jnp.mean(agg, axis=1) + skip_ref[...], 0.0)
    h3 = jnp.maximum(jnp.dot(h2, w3_ref[...], preferred_element_type=f32) + b3_ref[...], 0.0)
    out_ref[...] = jnp.dot(h3, w4_ref[...], preferred_element_type=f32) + b4_ref[...]


def _tc3(acc2, skip2, W3, b3, W4, b4):
    R = 1000
    grid = (N // R,)
    full = lambda a: pl.BlockSpec(a.shape, lambda i: (0,) * a.ndim)
    return pl.pallas_call(
        _tc3_body,
        grid=grid,
        in_specs=[pl.BlockSpec((NC, R, 64), lambda i: (0, i, 0)),
                  pl.BlockSpec((R, 8), lambda i: (i, 0)),
                  full(W3), full(b3), full(W4), full(b4)],
        out_specs=pl.BlockSpec((R, N_CLASSES), lambda i: (i, 0)),
        out_shape=jax.ShapeDtypeStruct((N, N_CLASSES), f32),
    )(acc2, skip2, W3, b3, W4, b4)


def kernel(x, edge_index, edge_attr,
           Wq1, bq1, Wk1, bk1, Wv1, bv1, We1, Wskip1, bskip1,
           Wq2, bq2, Wk2, bk2, Wv2, bv2, We2, Wskip2, bskip2,
           W3, b3, W4, b4):
    src = edge_index[0]
    dst = edge_index[1]

    qs1, kv1, skip1 = _tc1a(x, Wq1, bq1, Wk1, bk1, Wv1, bv1, Wskip1, bskip1)
    e1, e2 = _tc1b(edge_attr, We1, We2)

    acc1 = _sc_edge_pass(src, dst, qs1, kv1, e1, 80, 96, packed=False)
    qs2, kv2, skip2 = _tc2(acc1, skip1, Wq2, bq2, Wk2, bk2, Wv2, bv2,
                           Wskip2, bskip2)
    acc2 = _sc_edge_pass(src, dst, qs2, kv2, e2, 48, 64, packed=True)
    return _tc3(acc2, skip2, W3, b3, W4, b4)
